# Initial kernel scaffold; baseline (speedup 1.0000x reference)
import jax
import jax.numpy as jnp
from jax.experimental import pallas as pl


def kernel(points_features, points_neighbor, p1f, p1w, p1u, p2f, p2w, p2u, p3f, p3w, p3u):
    raise NotImplementedError("write your pallas kernel here")


# speedup vs baseline: 5.9765x; 5.9765x over previous
import functools

import jax
import jax.numpy as jnp
from jax import lax
from jax.experimental import pallas as pl
from jax.experimental.pallas import tpu as pltpu
from jax.experimental.pallas import tpu_sc as plsc

N = 100000
M = 9
E = N * M
BN = 2000
NB = N // BN
NC, NS = 2, 16
NW_USED = 30
BPW = E // NW_USED
EPS_BN = 1e-5
_CHUNK = {6: 3000, 12: 3000, 24: 1000, 48: 600}


def _fold(p):
    W1, b1, g1, be1, W2, b2, g2, be2 = p
    s = 1.0 / jnp.sqrt(1.0 + EPS_BN)
    a1 = g1 * s
    a2 = g2 * s
    W1e = W1.T * a1[None, :]
    b1e = (b1 * a1 + be1)[None, :]
    W2e = W2.T * a2[None, :]
    b2e = (b2 * a2 + be2)[None, :]
    return W1e, b1e, W2e, b2e


def _bd(wpair):
    W1e, b1e, W2e, b2e = wpair
    eye = jnp.eye(M, dtype=jnp.float32)
    return (jnp.kron(eye, W1e), jnp.tile(b1e, (1, M)),
            jnp.kron(eye, W2e), jnp.tile(b2e, (1, M)))


def _nb_body(n_ref, nb_ref):
    i = pl.program_id(0)
    n = n_ref[...]
    rows = lax.broadcasted_iota(jnp.int32, (BN, M), 0) + i * BN
    nb_ref[...] = jnp.where(n == 0, rows, n)


_nb_fix = pl.pallas_call(
    _nb_body,
    grid=(NB,),
    in_specs=[pl.BlockSpec((BN, M), lambda i: (i, 0))],
    out_specs=pl.BlockSpec((BN, M), lambda i: (i, 0)),
    out_shape=jax.ShapeDtypeStruct((N, M), jnp.int32),
)


def _prep_body(x_ref, w1e_ref, b1e_ref, w2e_ref, b2e_ref, pf6_ref, f1_ref,
               acc_ref):
    ph = pl.program_id(0)
    i = pl.program_id(1)
    x = x_ref[...]

    @pl.when((ph == 0) & (i == 0))
    def _():
        acc_ref[...] = jnp.zeros_like(acc_ref)

    @pl.when(ph == 0)
    def _():
        x3 = x[:, 0:3]
        acc_ref[...] = acc_ref[...] + jnp.sum(x3 * x3, axis=0, keepdims=True)

    @pl.when(ph == 1)
    def _():
        nrm = jnp.maximum(jnp.sqrt(acc_ref[...]), 1e-12)
        pf6 = jnp.concatenate([x[:, 0:3] / nrm, x[:, 3:6] * (1.0 / 255.0)],
                              axis=1)
        pf6_ref[...] = pf6
        h = jnp.maximum(
            jnp.dot(pf6, w1e_ref[...], preferred_element_type=jnp.float32)
            + b1e_ref[...], 0.0)
        f1_ref[...] = (
            jnp.dot(h, w2e_ref[...], preferred_element_type=jnp.float32)
            + b2e_ref[...])


def _full(shape):
    return pl.BlockSpec(shape, lambda ph, i: (0, 0))


_prep = pl.pallas_call(
    _prep_body,
    grid=(2, NB),
    in_specs=[
        pl.BlockSpec((BN, 9), lambda ph, i: (i, 0)),
        _full((6, 12)), _full((1, 12)), _full((12, 12)), _full((1, 12)),
    ],
    out_specs=[
        pl.BlockSpec((BN, 6), lambda ph, i: (i, 0)),
        pl.BlockSpec((BN, 12), lambda ph, i: (i, 0)),
    ],
    out_shape=[
        jax.ShapeDtypeStruct((N, 6), jnp.float32),
        jax.ShapeDtypeStruct((N, 12), jnp.float32),
    ],
    scratch_shapes=[pltpu.VMEM((1, 3), jnp.float32)],
)


def _make_sc_gather(D):
    CH = _CHUNK[D]
    nch = BPW // CH
    assert nch % 2 == 0 and CH % 8 == 0 and BPW % CH == 0
    mesh = plsc.VectorSubcoreMesh(core_axis_name="c", subcore_axis_name="s")

    @functools.partial(
        pl.kernel,
        mesh=mesh,
        compiler_params=pltpu.CompilerParams(use_tc_tiling_on_sc=False),
        out_type=jax.ShapeDtypeStruct((E, D), jnp.float32),
        scratch_types=[
            pltpu.VMEM((BPW,), jnp.int32),
            pltpu.VMEM((2, CH, D), jnp.float32),
            pltpu.SemaphoreType.DMA,
            pltpu.SemaphoreType.DMA,
            pltpu.SemaphoreType.DMA,
            pltpu.SemaphoreType.DMA,
        ],
    )
    def gk(table, idxh, out, idx_v, rows_v, g0, g1, w0, w1):
        wid = lax.axis_index("s") * NC + lax.axis_index("c")

        @pl.when(wid < NW_USED)
        def _():
            base = wid * BPW
            pltpu.sync_copy(idxh.at[pl.ds(base, BPW)], idx_v)
            gs = (g0, g1)
            ws = (w0, w1)

            def g_copy(k, slot):
                return pltpu.make_async_copy(
                    table.at[idx_v.at[pl.ds(k * CH, CH)]], rows_v.at[slot],
                    gs[slot])

            def w_copy(k, slot):
                return pltpu.make_async_copy(
                    rows_v.at[slot], out.at[pl.ds(base + k * CH, CH)],
                    ws[slot])

            g_copy(0, 0).start()
            g_copy(1, 1).start()

            def loop_body(hh, carry):
                k0 = 2 * hh
                k1 = k0 + 1
                g_copy(k0, 0).wait()
                w_copy(k0, 0).start()
                g_copy(k1, 1).wait()
                w_copy(k1, 1).start()
                w_copy(k0, 0).wait()

                @pl.when(k0 + 2 < nch)
                def _():
                    g_copy(k0 + 2, 0).start()

                w_copy(k1, 1).wait()

                @pl.when(k1 + 2 < nch)
                def _():
                    g_copy(k1 + 2, 1).start()

                return carry

            lax.fori_loop(0, nch // 2, loop_body, 0)

    return gk


_gather6 = _make_sc_gather(6)
_gather12 = _make_sc_gather(12)
_gather24 = _make_sc_gather(24)
_gather48 = _make_sc_gather(48)


def _make_combiner(C, Cn, final):
    d9c = M * C

    def body(nbr_ref, sel_ref, g_ref, wbd1_ref, bbd1_ref, wbd2_ref, bbd2_ref,
             wu1_ref, bu1_ref, wu2_ref, bu2_ref, *rest):
        if final:
            feas1_ref, feas2_ref, pf6_ref, out_ref = rest
        else:
            wf1_ref, bf1_ref, wf2_ref, bf2_ref, feas_ref, fn_ref = rest
        sel = sel_ref[...]
        xyz = nbr_ref[...] - jnp.concatenate([sel] * M, axis=1)
        h = jnp.maximum(
            jnp.dot(xyz, wbd1_ref[...], preferred_element_type=jnp.float32)
            + bbd1_ref[...], 0.0)
        w = (jnp.dot(h, wbd2_ref[...], preferred_element_type=jnp.float32)
             + bbd2_ref[...])
        prod = w * g_ref[...]
        t = jnp.maximum(
            jnp.dot(prod, wu1_ref[...], preferred_element_type=jnp.float32)
            + bu1_ref[...], 0.0)
        feas = (jnp.dot(t, wu2_ref[...], preferred_element_type=jnp.float32)
                + bu2_ref[...])
        if final:
            out_ref[...] = jnp.concatenate(
                [feas, feas2_ref[...], feas1_ref[...], pf6_ref[...]], axis=1)
        else:
            feas_ref[...] = feas
            th = jnp.maximum(
                jnp.dot(feas, wf1_ref[...],
                        preferred_element_type=jnp.float32) + bf1_ref[...],
                0.0)
            fn_ref[...] = (
                jnp.dot(th, wf2_ref[...], preferred_element_type=jnp.float32)
                + bf2_ref[...])

    def blk(shape):
        return pl.BlockSpec(shape, lambda i: (i, 0))

    def wfull(shape):
        return pl.BlockSpec(shape, lambda i: (0, 0))

    in_specs = [
        blk((BN, 6 * M)), blk((BN, 6)), blk((BN, d9c)),
        wfull((6 * M, d9c)), wfull((1, d9c)), wfull((d9c, d9c)),
        wfull((1, d9c)),
        wfull((d9c, C)), wfull((1, C)), wfull((C, C)), wfull((1, C)),
    ]
    if final:
        in_specs += [blk((BN, 12)), blk((BN, 24)), blk((BN, 6))]
        out_specs = blk((BN, 90))
        out_shape = jax.ShapeDtypeStruct((N, 90), jnp.float32)
    else:
        in_specs += [wfull((C, Cn)), wfull((1, Cn)), wfull((Cn, Cn)),
                     wfull((1, Cn))]
        out_specs = [blk((BN, C)), blk((BN, Cn))]
        out_shape = [jax.ShapeDtypeStruct((N, C), jnp.float32),
                     jax.ShapeDtypeStruct((N, Cn), jnp.float32)]

    return pl.pallas_call(body, grid=(NB,), in_specs=in_specs,
                          out_specs=out_specs, out_shape=out_shape)


_comb1 = _make_combiner(12, 24, final=False)
_comb2 = _make_combiner(24, 48, final=False)
_comb3 = _make_combiner(48, None, final=True)


def kernel(points_features, points_neighbor, p1f, p1w, p1u, p2f, p2w, p2u,
           p3f, p3w, p3u):
    x = points_features
    sel = jnp.concatenate([x[:, 0:3], x[:, 6:9]], axis=1)
    idx = _nb_fix(points_neighbor).reshape(E)

    f1w = _fold(p1f)
    u1w = _fold(p1u)
    f2w = _fold(p2f)
    u2w = _fold(p2u)
    u3w = _fold(p3u)
    wbd1 = _bd(_fold(p1w))
    wbd2 = _bd(_fold(p2w))
    wbd3 = _bd(_fold(p3w))

    nbr54 = _gather6(sel, idx).reshape(N, 6 * M)
    pf6, f1 = _prep(x, *f1w)
    g1 = _gather12(f1, idx).reshape(N, 12 * M)
    feas1, f2 = _comb1(nbr54, sel, g1, *wbd1, *u1w, *f2w)
    g2 = _gather24(f2, idx).reshape(N, 24 * M)
    feas2, f3 = _comb2(nbr54, sel, g2, *wbd2, *u2w, *_fold(p3f))
    g3 = _gather48(f3, idx).reshape(N, 48 * M)
    return _comb3(nbr54, sel, g3, *wbd3, *u3w, feas1, feas2, pf6)


# speedup vs baseline: 7.5056x; 1.2558x over previous
import functools

import jax
import jax.numpy as jnp
from jax import lax
from jax.experimental import pallas as pl
from jax.experimental.pallas import tpu as pltpu
from jax.experimental.pallas import tpu_sc as plsc

N = 100000
M = 9
MG = 10
E = N * MG
BN = 2000
NB = N // BN
NC, NS = 2, 16
NW_USED = 25
BPW = E // NW_USED
EPS_BN = 1e-5
_CHUNK = {24: 1000, 32: 800, 56: 400}


def _fold(p):
    W1, b1, g1, be1, W2, b2, g2, be2 = p
    s = 1.0 / jnp.sqrt(1.0 + EPS_BN)
    a1 = g1 * s
    a2 = g2 * s
    W1e = W1.T * a1[None, :]
    b1e = (b1 * a1 + be1)[None, :]
    W2e = W2.T * a2[None, :]
    b2e = (b2 * a2 + be2)[None, :]
    return W1e, b1e, W2e, b2e


def _bd_ext(wpair, C, D1):
    W1e, b1e, W2e, b2e = wpair
    eye = jnp.eye(M, dtype=jnp.float32)
    W1ext = jnp.concatenate([W1e, jnp.zeros((D1 - 6, C), jnp.float32)],
                             axis=0)
    return (jnp.kron(eye, W1ext), jnp.tile(b1e, (1, M)),
            jnp.kron(eye, W2e), jnp.tile(b2e, (1, M)), W1e)


def _nb_body(n_ref, x_ref, nb_ref, sums_ref, acc_ref):
    i = pl.program_id(0)
    n = n_ref[...]
    rows = lax.broadcasted_iota(jnp.int32, (BN, M), 0) + i * BN
    rowcol = lax.broadcasted_iota(jnp.int32, (BN, 1), 0) + i * BN
    nb_ref[...] = jnp.concatenate([jnp.where(n == 0, rows, n), rowcol],
                                  axis=1)

    @pl.when(i == 0)
    def _():
        acc_ref[...] = jnp.zeros_like(acc_ref)

    x3 = x_ref[:, 0:3]
    acc_ref[...] = acc_ref[...] + jnp.sum(x3 * x3, axis=0, keepdims=True)
    sums_ref[...] = acc_ref[...]


_nb_fix = pl.pallas_call(
    _nb_body,
    grid=(NB,),
    in_specs=[pl.BlockSpec((BN, M), lambda i: (i, 0)),
              pl.BlockSpec((BN, 9), lambda i: (i, 0))],
    out_specs=[pl.BlockSpec((BN, MG), lambda i: (i, 0)),
               pl.BlockSpec((1, 3), lambda i: (0, 0))],
    out_shape=[jax.ShapeDtypeStruct((N, MG), jnp.int32),
               jax.ShapeDtypeStruct((1, 3), jnp.float32)],
    scratch_shapes=[pltpu.VMEM((1, 3), jnp.float32)],
)


def _prep_body(x_ref, sums_ref, w1e_ref, b1e_ref, w2e_ref, b2e_ref,
               pf6_ref, t1_ref):
    x = x_ref[...]
    nrm = jnp.maximum(jnp.sqrt(sums_ref[...]), 1e-12)
    pf6 = jnp.concatenate([x[:, 0:3] / nrm, x[:, 3:6] * (1.0 / 255.0)],
                          axis=1)
    pf6_ref[...] = pf6
    h = jnp.maximum(
        jnp.dot(pf6, w1e_ref[...], preferred_element_type=jnp.float32)
        + b1e_ref[...], 0.0)
    f1 = (jnp.dot(h, w2e_ref[...], preferred_element_type=jnp.float32)
          + b2e_ref[...])
    sel = jnp.concatenate([x[:, 0:3], x[:, 6:9]], axis=1)
    t1_ref[...] = jnp.concatenate(
        [sel, f1, jnp.zeros((sel.shape[0], 6), jnp.float32)], axis=1)


def _wfull(shape):
    return pl.BlockSpec(shape, lambda i: (0, 0))


_prep = pl.pallas_call(
    _prep_body,
    grid=(NB,),
    in_specs=[
        pl.BlockSpec((BN, 9), lambda i: (i, 0)),
        _wfull((1, 3)),
        _wfull((6, 12)), _wfull((1, 12)), _wfull((12, 12)), _wfull((1, 12)),
    ],
    out_specs=[
        pl.BlockSpec((BN, 6), lambda i: (i, 0)),
        pl.BlockSpec((BN, 24), lambda i: (i, 0)),
    ],
    out_shape=[
        jax.ShapeDtypeStruct((N, 6), jnp.float32),
        jax.ShapeDtypeStruct((N, 24), jnp.float32),
    ],
)


def _make_sc_gather(D):
    CH = _CHUNK[D]
    nch = BPW // CH
    assert nch % 2 == 0 and CH % 8 == 0 and BPW % CH == 0
    mesh = plsc.VectorSubcoreMesh(core_axis_name="c", subcore_axis_name="s")

    @functools.partial(
        pl.kernel,
        mesh=mesh,
        compiler_params=pltpu.CompilerParams(use_tc_tiling_on_sc=False),
        out_type=jax.ShapeDtypeStruct((E, D), jnp.float32),
        scratch_types=[
            pltpu.VMEM((BPW,), jnp.int32),
            pltpu.VMEM((2, CH, D), jnp.float32),
            pltpu.SemaphoreType.DMA,
            pltpu.SemaphoreType.DMA,
            pltpu.SemaphoreType.DMA,
            pltpu.SemaphoreType.DMA,
        ],
    )
    def gk(table, idxh, out, idx_v, rows_v, g0, g1, w0, w1):
        wid = lax.axis_index("s") * NC + lax.axis_index("c")

        @pl.when(wid < NW_USED)
        def _():
            base = wid * BPW
            pltpu.sync_copy(idxh.at[pl.ds(base, BPW)], idx_v)
            gs = (g0, g1)
            ws = (w0, w1)

            def g_copy(k, slot):
                return pltpu.make_async_copy(
                    table.at[idx_v.at[pl.ds(k * CH, CH)]], rows_v.at[slot],
                    gs[slot])

            def w_copy(k, slot):
                return pltpu.make_async_copy(
                    rows_v.at[slot], out.at[pl.ds(base + k * CH, CH)],
                    ws[slot])

            g_copy(0, 0).start()
            g_copy(1, 1).start()

            def loop_body(hh, carry):
                k0 = 2 * hh
                k1 = k0 + 1
                g_copy(k0, 0).wait()
                w_copy(k0, 0).start()
                g_copy(k1, 1).wait()
                w_copy(k1, 1).start()
                w_copy(k0, 0).wait()

                @pl.when(k0 + 2 < nch)
                def _():
                    g_copy(k0 + 2, 0).start()

                w_copy(k1, 1).wait()

                @pl.when(k1 + 2 < nch)
                def _():
                    g_copy(k1 + 2, 1).start()

                return carry

            lax.fori_loop(0, nch // 2, loop_body, 0)

    return gk


_gather24 = _make_sc_gather(24)
_gather32 = _make_sc_gather(32)
_gather56 = _make_sc_gather(56)


def _make_combiner(C, Cn, D1, D1n, final):
    d9c = M * C

    def body(g_ref, wbd1_ref, bbd1_ref, wbd2_ref, bbd2_ref, wself_ref,
             wu1_ref, bu1_ref, wu2_ref, bu2_ref, *rest):
        if final:
            feas1_ref, feas2_ref, pf6_ref, out_ref = rest
        else:
            wf1_ref, bf1_ref, wf2_ref, bf2_ref, feas_ref, tn_ref = rest
        g = g_ref[...]
        s = g[:, M * D1:M * D1 + 6]
        selw = jnp.dot(s, wself_ref[...], preferred_element_type=jnp.float32)
        selw9 = jnp.concatenate([selw] * M, axis=1)
        h = jnp.maximum(
            jnp.dot(g[:, :M * D1], wbd1_ref[...],
                    preferred_element_type=jnp.float32)
            + bbd1_ref[...] - selw9, 0.0)
        w = (jnp.dot(h, wbd2_ref[...], preferred_element_type=jnp.float32)
             + bbd2_ref[...])
        gf = jnp.concatenate(
            [g[:, m * D1 + 6:m * D1 + 6 + C] for m in range(M)], axis=1)
        prod = w * gf
        t = jnp.maximum(
            jnp.dot(prod, wu1_ref[...], preferred_element_type=jnp.float32)
            + bu1_ref[...], 0.0)
        feas = (jnp.dot(t, wu2_ref[...], preferred_element_type=jnp.float32)
                + bu2_ref[...])
        if final:
            out_ref[...] = jnp.concatenate(
                [feas, feas2_ref[...], feas1_ref[...], pf6_ref[...]], axis=1)
        else:
            feas_ref[...] = feas
            th = jnp.maximum(
                jnp.dot(feas, wf1_ref[...],
                        preferred_element_type=jnp.float32) + bf1_ref[...],
                0.0)
            fn = (jnp.dot(th, wf2_ref[...],
                          preferred_element_type=jnp.float32) + bf2_ref[...])
            pad = jnp.zeros((s.shape[0], D1n - 6 - Cn), jnp.float32)
            tn_ref[...] = jnp.concatenate([s, fn, pad], axis=1)

    def blk(shape):
        return pl.BlockSpec(shape, lambda i: (i, 0))

    in_specs = [
        blk((BN, MG * D1)),
        _wfull((M * D1, d9c)), _wfull((1, d9c)), _wfull((d9c, d9c)),
        _wfull((1, d9c)), _wfull((6, C)),
        _wfull((d9c, C)), _wfull((1, C)), _wfull((C, C)), _wfull((1, C)),
    ]
    if final:
        in_specs += [blk((BN, 12)), blk((BN, 24)), blk((BN, 6))]
        out_specs = blk((BN, 90))
        out_shape = jax.ShapeDtypeStruct((N, 90), jnp.float32)
    else:
        in_specs += [_wfull((C, Cn)), _wfull((1, Cn)), _wfull((Cn, Cn)),
                     _wfull((1, Cn))]
        out_specs = [blk((BN, C)), blk((BN, D1n))]
        out_shape = [jax.ShapeDtypeStruct((N, C), jnp.float32),
                     jax.ShapeDtypeStruct((N, D1n), jnp.float32)]

    return pl.pallas_call(body, grid=(NB,), in_specs=in_specs,
                          out_specs=out_specs, out_shape=out_shape)


_comb1 = _make_combiner(12, 24, 24, 32, final=False)
_comb2 = _make_combiner(24, 48, 32, 56, final=False)
_comb3 = _make_combiner(48, None, 56, None, final=True)


def kernel(points_features, points_neighbor, p1f, p1w, p1u, p2f, p2w, p2u,
           p3f, p3w, p3u):
    x = points_features

    nb10, sums = _nb_fix(points_neighbor, x)
    idx = nb10.reshape(E)

    f1w = _fold(p1f)
    u1w = _fold(p1u)
    f2w = _fold(p2f)
    u2w = _fold(p2u)
    f3w = _fold(p3f)
    u3w = _fold(p3u)
    wbd1 = _bd_ext(_fold(p1w), 12, 24)
    wbd2 = _bd_ext(_fold(p2w), 24, 32)
    wbd3 = _bd_ext(_fold(p3w), 48, 56)

    pf6, t1 = _prep(x, sums, *f1w)
    g1 = _gather24(t1, idx).reshape(N, MG * 24)
    feas1, t2 = _comb1(g1, *wbd1, *u1w, *f2w)
    g2 = _gather32(t2, idx).reshape(N, MG * 32)
    feas2, t3 = _comb2(g2, *wbd2, *u2w, *f3w)
    g3 = _gather56(t3, idx).reshape(N, MG * 56)
    return _comb3(g3, *wbd3, *u3w, feas1, feas2, pf6)
